# baseline (speedup 1.0000x reference)
import functools

import jax
import jax.numpy as jnp
from jax import lax
from jax.experimental import pallas as pl
from jax.experimental.pallas import tpu as pltpu
from jax.experimental.pallas import tpu_sc as plsc

VSZ = 1000000
DSZ = 64
MXLEN = 512
B = 4096
T = 200

_info = plsc.get_sparse_core_info()
_NC, _NS, _L = _info.num_cores, _info.num_subcores, _info.num_lanes
_NW = _NC * _NS
_ROWS_PER_W = B // _NW
_HALF = T // 2
_NBUF = 4


def _body(x_hbm, w_hbm, pos_hbm, out_hbm,
          idx_v, pos_v, bufs, gsems, wsems):
    wid = lax.axis_index("s") * _NC + lax.axis_index("c")
    b0 = wid * _ROWS_PER_W

    pltpu.sync_copy(x_hbm.at[pl.ds(b0, _ROWS_PER_W)], idx_v)
    pltpu.sync_copy(pos_hbm.at[pl.ds(0, T)], pos_v)

    def fire_gather(r, k):
        pltpu.async_copy(w_hbm.at[idx_v.at[r, 0]],
                         bufs[k].at[pl.ds(0, _HALF)], gsems[k])
        pltpu.async_copy(w_hbm.at[idx_v.at[r, 1]],
                         bufs[k].at[pl.ds(_HALF, _HALF)], gsems[k])

    def wait_gather(r, k):
        pltpu.make_async_copy(w_hbm.at[idx_v.at[r, 0]],
                              bufs[k].at[pl.ds(0, _HALF)], gsems[k]).wait()
        pltpu.make_async_copy(w_hbm.at[idx_v.at[r, 1]],
                              bufs[k].at[pl.ds(_HALF, _HALF)], gsems[k]).wait()

    def wait_write(r, k):
        pltpu.make_async_copy(bufs[k], out_hbm.at[b0 + r], wsems[k]).wait()

    def b_step(r, k):
        wait_gather(r, k)
        buf = bufs[k]

        def tok_body(i, c):
            for tloc in range(8):
                tok = i * 8 + tloc
                for kk in range(DSZ // _L):
                    sl = pl.ds(kk * _L, _L)
                    plsc.addupdate(buf.at[tok, sl], pos_v[tok, sl])
            return c

        lax.fori_loop(0, T // 8, tok_body, 0)
        pltpu.async_copy(buf, out_hbm.at[b0 + r], wsems[k])

    def prep(r, k, with_wait):
        if with_wait:
            wait_write(r - _NBUF, k)
        fire_gather(r, k)

    prep(0, 0, False)
    prep(1, 1, False)
    prep(2, 2, False)
    b_step(0, 0)
    prep(3, 3, False)
    b_step(1, 1)

    def outer(g, c):
        r0 = g * _NBUF
        for k in range(_NBUF):
            r = r0 + k
            prep(r + 2, (k + 2) % _NBUF, True)
            b_step(r, k)
        return c

    prep(4, 0, True)
    b_step(2, 2)
    prep(5, 1, True)
    b_step(3, 3)
    lax.fori_loop(1, (_ROWS_PER_W // _NBUF) - 1, outer, 0)
    rlast = _ROWS_PER_W - _NBUF
    prep(rlast + 2, 2, True)
    b_step(rlast + 0, 0)
    prep(rlast + 3, 3, True)
    b_step(rlast + 1, 1)
    b_step(rlast + 2, 2)
    b_step(rlast + 3, 3)
    for k in range(_NBUF):
        wait_write(rlast + k, k)


def kernel(x, W, pos):
    x3 = x.reshape(B, 2, _HALF)
    mesh = plsc.VectorSubcoreMesh(core_axis_name="c", subcore_axis_name="s")
    fn = functools.partial(
        pl.kernel,
        mesh=mesh,
        out_type=jax.ShapeDtypeStruct((B, T, DSZ), jnp.float32),
        scratch_types=[
            pltpu.VMEM((_ROWS_PER_W, 2, _HALF), jnp.int32),
            pltpu.VMEM((T, DSZ), jnp.float32),
            [pltpu.VMEM((T, DSZ), jnp.float32) for _ in range(_NBUF)],
            [pltpu.SemaphoreType.DMA for _ in range(_NBUF)],
            [pltpu.SemaphoreType.DMA for _ in range(_NBUF)],
        ],
        compiler_params=pltpu.CompilerParams(use_tc_tiling_on_sc=False),
    )(_body)
    return fn(x3, W, pos)
